# baseline (speedup 1.0000x reference)
import functools

import jax
import jax.numpy as jnp
from jax import lax
from jax.experimental import pallas as pl
from jax.experimental.pallas import tpu as pltpu

B, T, CIN = 8, 2048, 1
H, L, E, K, NH, F, PL_ = 1024, 2, 8, 2, 16, 1024, 32
NP = T // PL_
N = B * NP
CAP = int(1.25 * N * K / E)
DH = H // NH


def _rms(x, w):
    return x * w * lax.rsqrt(jnp.mean(x * x, axis=-1, keepdims=True) + 1e-6)


def _embed_body(p_ref, gw_ref, ew_ref, h_ref):
    p = p_ref[...]
    h_ref[...] = jax.nn.silu(p @ gw_ref[...]) * (p @ ew_ref[...])


def _attn_body(x_ref, ln1_ref, qkv_ref, o_ref, ln2_ref, hs_ref):
    xb = x_ref[0]
    ni = _rms(xb, ln1_ref[...])
    qkv = ni @ qkv_ref[...]
    r_iota = lax.broadcasted_iota(jnp.int32, (NP, NP), 0)
    c_iota = lax.broadcasted_iota(jnp.int32, (NP, NP), 1)
    causal = r_iota >= c_iota
    ctxs = []
    for h in range(NH):
        q = qkv[:, h * DH:(h + 1) * DH]
        k = qkv[:, H + h * DH:H + (h + 1) * DH]
        v = qkv[:, 2 * H + h * DH:2 * H + (h + 1) * DH]
        s = jnp.dot(q, k.T) * (1.0 / (DH ** 0.5))
        s = jnp.where(causal, s, -1e9)
        s = s - jnp.max(s, axis=-1, keepdims=True)
        p = jnp.exp(s)
        a = p / jnp.sum(p, axis=-1, keepdims=True)
        ctxs.append(jnp.dot(a, v))
    ctx = jnp.concatenate(ctxs, axis=-1)
    attn = ctx @ o_ref[...]
    hs_ref[0] = _rms(attn + ni, ln2_ref[...])


def _route_body(hs_ref, wr_ref, meta_ref, stats_ref):
    hs = hs_ref[...]
    logits = hs @ wr_ref[...]
    m = jnp.max(logits, axis=-1, keepdims=True)
    ex = jnp.exp(logits - m)
    probs = ex / jnp.sum(ex, axis=-1, keepdims=True)

    e_iota = lax.broadcasted_iota(jnp.int32, (N, E), 1)
    m0 = jnp.max(probs, axis=-1, keepdims=True)
    i0 = jnp.min(jnp.where(probs == m0, e_iota, E), axis=-1, keepdims=True)
    oh0 = (e_iota == i0).astype(jnp.float32)
    probs1 = jnp.where(e_iota == i0, -1.0, probs)
    m1 = jnp.max(probs1, axis=-1, keepdims=True)
    i1 = jnp.min(jnp.where(probs1 == m1, e_iota, E), axis=-1, keepdims=True)
    oh1 = (e_iota == i1).astype(jnp.float32)

    denom = m0 + m1 + 1e-9
    g0 = m0 / denom
    g1 = m1 / denom

    lt = (lax.broadcasted_iota(jnp.int32, (N, N), 0)
          > lax.broadcasted_iota(jnp.int32, (N, N), 1)).astype(jnp.float32)
    cum0 = jnp.dot(lt, oh0)
    pos0 = jnp.sum(cum0 * oh0, axis=-1, keepdims=True)
    total0 = jnp.sum(oh0, axis=0, keepdims=True)
    cum1 = jnp.dot(lt, oh1) + total0
    pos1 = jnp.sum(cum1 * oh1, axis=-1, keepdims=True)

    keep0 = (pos0 < CAP).astype(jnp.float32)
    keep1 = (pos1 < CAP).astype(jnp.float32)
    stats_ref[...] = jnp.sum(keep0 * oh0 + keep1 * oh1, axis=0, keepdims=True)

    zero = jnp.zeros((N, 1), jnp.float32)
    meta_ref[...] = jnp.concatenate(
        [i0.astype(jnp.float32), pos0, keep0 * g0,
         i1.astype(jnp.float32), pos1, keep1 * g1, zero, zero], axis=-1)


def _moe_body(hs_ref, meta_ref, w1_ref, w2_ref, out_ref):
    e = pl.program_id(0)
    hs = hs_ref[...]
    meta = meta_ref[...]
    kg0, kg1 = meta[:, 2:3], meta[:, 5:6]
    i0 = meta[:, 0:1].astype(jnp.int32)
    i1 = meta[:, 3:4].astype(jnp.int32)
    pos0 = meta[:, 1:2].astype(jnp.int32)
    pos1 = meta[:, 4:5].astype(jnp.int32)
    c_iota = lax.broadcasted_iota(jnp.int32, (N, CAP), 1)
    d0 = ((i0 == e) & (pos0 == c_iota)).astype(jnp.float32)
    d1 = ((i1 == e) & (pos1 == c_iota)).astype(jnp.float32)
    disp = d0 + d1
    x_e = lax.dot_general(disp, hs, (((0,), (0,)), ((), ())))
    hmid = jax.nn.silu(x_e @ w1_ref[0])
    eout = hmid @ w2_ref[0]
    comb = kg0 * d0 + kg1 * d1
    y = jnp.dot(comb, eout)

    @pl.when(e == 0)
    def _():
        out_ref[...] = 2.0 * hs + y

    @pl.when(e != 0)
    def _():
        out_ref[...] += y


def _heads_body(h_ref, w_ref, b_ref, out_ref):
    out_ref[...] = h_ref[...] @ w_ref[...] + b_ref[...]


def _full(shape):
    return pl.BlockSpec(shape, lambda *_: tuple(0 for _ in shape))


def kernel(x, emb_W, gate_W, ln1_w, qkv_W, o_W, ln2_w, router_W,
           exp_W1, exp_W2, hW1, hb1, hW8, hb8, hW32, hb32, hW64, hb64):
    f32 = jnp.float32
    p = x.reshape(B, NP, PL_, CIN).transpose(0, 1, 3, 2).reshape(N, CIN * PL_)

    h = pl.pallas_call(
        _embed_body,
        out_shape=jax.ShapeDtypeStruct((N, H), f32),
    )(p, gate_W, emb_W)

    attn_call = pl.pallas_call(
        _attn_body,
        grid=(B,),
        in_specs=[
            pl.BlockSpec((1, NP, H), lambda b: (b, 0, 0)),
            pl.BlockSpec((1, H), lambda b: (0, 0)),
            pl.BlockSpec((H, 3 * H), lambda b: (0, 0)),
            pl.BlockSpec((H, H), lambda b: (0, 0)),
            pl.BlockSpec((1, H), lambda b: (0, 0)),
        ],
        out_specs=pl.BlockSpec((1, NP, H), lambda b: (b, 0, 0)),
        out_shape=jax.ShapeDtypeStruct((B, NP, H), f32),
    )

    route_call = pl.pallas_call(
        _route_body,
        out_shape=(jax.ShapeDtypeStruct((N, 8), f32),
                   jax.ShapeDtypeStruct((1, E), f32)),
    )

    moe_call = pl.pallas_call(
        _moe_body,
        grid=(E,),
        in_specs=[
            _full((N, H)),
            _full((N, 8)),
            pl.BlockSpec((1, H, F), lambda e: (e, 0, 0)),
            pl.BlockSpec((1, F, H), lambda e: (e, 0, 0)),
        ],
        out_specs=_full((N, H)),
        out_shape=jax.ShapeDtypeStruct((N, H), f32),
    )

    stats = jnp.zeros((E,), f32)
    for l in range(L):
        hs = attn_call(h.reshape(B, NP, H), ln1_w[l:l + 1],
                       qkv_W[l], o_W[l], ln2_w[l:l + 1]).reshape(N, H)
        meta, st = route_call(hs, router_W[l])
        h = moe_call(hs, meta, exp_W1[l], exp_W2[l])
        stats = stats + st[0]

    hz = [1, 8, 32, 64]
    Wh = jnp.concatenate([hW1, hW8, hW32, hW64], axis=1)
    bh = jnp.concatenate([hb1, hb8, hb32, hb64]).reshape(1, -1)
    outs = pl.pallas_call(
        _heads_body,
        out_shape=jax.ShapeDtypeStruct((N, sum(hz)), f32),
    )(h, Wh, bh)

    res, off = [], 0
    for z in hz:
        res.append(outs[:, off:off + z].reshape(B, NP, z))
        off += z
    return (*res, stats)


# speedup vs baseline: 1.3641x; 1.3641x over previous
import functools

import jax
import jax.numpy as jnp
from jax import lax
from jax.experimental import pallas as pl
from jax.experimental.pallas import tpu as pltpu

B, T, CIN = 8, 2048, 1
H, L, E, K, NH, F, PL_ = 1024, 2, 8, 2, 16, 1024, 32
NP = T // PL_
N = B * NP
CAP = int(1.25 * N * K / E)
DH = H // NH


def _rms(x, w):
    return x * w * lax.rsqrt(jnp.mean(x * x, axis=-1, keepdims=True) + 1e-6)


def _embed_body(p_ref, gw_ref, ew_ref, h_ref):
    p = p_ref[...]
    h_ref[...] = jax.nn.silu(p @ gw_ref[...]) * (p @ ew_ref[...])


def _attnroute_body(x_ref, ln1_ref, qkv_ref, o_ref, ln2_ref, wr_ref,
                    hs_ref, meta_ref, stats_ref):
    x = x_ref[...]
    ni = _rms(x, ln1_ref[...])
    qkv = ni @ qkv_ref[...]
    r_iota = lax.broadcasted_iota(jnp.int32, (N, N), 0)
    c_iota = lax.broadcasted_iota(jnp.int32, (N, N), 1)
    mask = ((r_iota // NP) == (c_iota // NP)) & (r_iota >= c_iota)
    ctxs = []
    for h in range(NH):
        q = qkv[:, h * DH:(h + 1) * DH]
        k = qkv[:, H + h * DH:H + (h + 1) * DH]
        v = qkv[:, 2 * H + h * DH:2 * H + (h + 1) * DH]
        s = lax.dot_general(q, k, (((1,), (1,)), ((), ())))
        s = jnp.where(mask, s * (1.0 / (DH ** 0.5)), -1e9)
        s = s - jnp.max(s, axis=-1, keepdims=True)
        p = jnp.exp(s)
        a = p / jnp.sum(p, axis=-1, keepdims=True)
        ctxs.append(jnp.dot(a, v))
    ctx = jnp.concatenate(ctxs, axis=-1)
    attn = ctx @ o_ref[...]
    hs = _rms(attn + ni, ln2_ref[...])
    hs_ref[...] = hs

    logits = hs @ wr_ref[...]
    m = jnp.max(logits, axis=-1, keepdims=True)
    ex = jnp.exp(logits - m)
    probs = ex / jnp.sum(ex, axis=-1, keepdims=True)

    e_iota = lax.broadcasted_iota(jnp.int32, (N, E), 1)
    m0 = jnp.max(probs, axis=-1, keepdims=True)
    i0 = jnp.min(jnp.where(probs == m0, e_iota, E), axis=-1, keepdims=True)
    oh0 = (e_iota == i0).astype(jnp.float32)
    probs1 = jnp.where(e_iota == i0, -1.0, probs)
    m1 = jnp.max(probs1, axis=-1, keepdims=True)
    i1 = jnp.min(jnp.where(probs1 == m1, e_iota, E), axis=-1, keepdims=True)
    oh1 = (e_iota == i1).astype(jnp.float32)

    denom = m0 + m1 + 1e-9
    g0 = m0 / denom
    g1 = m1 / denom

    lt = (lax.broadcasted_iota(jnp.int32, (N, N), 0)
          > lax.broadcasted_iota(jnp.int32, (N, N), 1)).astype(jnp.float32)
    cum0 = jnp.dot(lt, oh0)
    pos0 = jnp.sum(cum0 * oh0, axis=-1, keepdims=True)
    total0 = jnp.sum(oh0, axis=0, keepdims=True)
    cum1 = jnp.dot(lt, oh1) + total0
    pos1 = jnp.sum(cum1 * oh1, axis=-1, keepdims=True)

    keep0 = (pos0 < CAP).astype(jnp.float32)
    keep1 = (pos1 < CAP).astype(jnp.float32)
    stats_ref[...] = jnp.sum(keep0 * oh0 + keep1 * oh1, axis=0, keepdims=True)

    zero = jnp.zeros((N, 1), jnp.float32)
    meta_ref[...] = jnp.concatenate(
        [i0.astype(jnp.float32), pos0, keep0 * g0,
         i1.astype(jnp.float32), pos1, keep1 * g1, zero, zero], axis=-1)


def _moe_body(hs_ref, meta_ref, w1_ref, w2_ref, out_ref):
    e = pl.program_id(0)
    hs = hs_ref[...]
    meta = meta_ref[...]
    kg0, kg1 = meta[:, 2:3], meta[:, 5:6]
    i0 = meta[:, 0:1].astype(jnp.int32)
    i1 = meta[:, 3:4].astype(jnp.int32)
    pos0 = meta[:, 1:2].astype(jnp.int32)
    pos1 = meta[:, 4:5].astype(jnp.int32)
    c_iota = lax.broadcasted_iota(jnp.int32, (N, CAP), 1)
    d0 = ((i0 == e) & (pos0 == c_iota)).astype(jnp.float32)
    d1 = ((i1 == e) & (pos1 == c_iota)).astype(jnp.float32)
    disp = d0 + d1
    x_e = lax.dot_general(disp, hs, (((0,), (0,)), ((), ())))
    hmid = jax.nn.silu(x_e @ w1_ref[0])
    eout = hmid @ w2_ref[0]
    comb = kg0 * d0 + kg1 * d1
    y = jnp.dot(comb, eout)

    @pl.when(e == 0)
    def _():
        out_ref[...] = 2.0 * hs + y

    @pl.when(e != 0)
    def _():
        out_ref[...] += y


def _heads_body(h_ref, w_ref, b_ref, out_ref):
    out_ref[...] = h_ref[...] @ w_ref[...] + b_ref[...]


def _full(shape):
    return pl.BlockSpec(shape, lambda *_: tuple(0 for _ in shape))


def kernel(x, emb_W, gate_W, ln1_w, qkv_W, o_W, ln2_w, router_W,
           exp_W1, exp_W2, hW1, hb1, hW8, hb8, hW32, hb32, hW64, hb64):
    f32 = jnp.float32
    p = x.reshape(B, NP, PL_, CIN).transpose(0, 1, 3, 2).reshape(N, CIN * PL_)

    h = pl.pallas_call(
        _embed_body,
        out_shape=jax.ShapeDtypeStruct((N, H), f32),
    )(p, gate_W, emb_W)

    attnroute_call = pl.pallas_call(
        _attnroute_body,
        out_shape=(jax.ShapeDtypeStruct((N, H), f32),
                   jax.ShapeDtypeStruct((N, 8), f32),
                   jax.ShapeDtypeStruct((1, E), f32)),
    )

    moe_call = pl.pallas_call(
        _moe_body,
        grid=(E,),
        in_specs=[
            _full((N, H)),
            _full((N, 8)),
            pl.BlockSpec((1, H, F), lambda e: (e, 0, 0)),
            pl.BlockSpec((1, F, H), lambda e: (e, 0, 0)),
        ],
        out_specs=_full((N, H)),
        out_shape=jax.ShapeDtypeStruct((N, H), f32),
    )

    stats = jnp.zeros((E,), f32)
    for l in range(L):
        hs, meta, st = attnroute_call(h, ln1_w[l:l + 1], qkv_W[l], o_W[l],
                                      ln2_w[l:l + 1], router_W[l])
        h = moe_call(hs, meta, exp_W1[l], exp_W2[l])
        stats = stats + st[0]

    hz = [1, 8, 32, 64]
    Wh = jnp.concatenate([hW1, hW8, hW32, hW64], axis=1)
    bh = jnp.concatenate([hb1, hb8, hb32, hb64]).reshape(1, -1)
    outs = pl.pallas_call(
        _heads_body,
        out_shape=jax.ShapeDtypeStruct((N, sum(hz)), f32),
    )(h, Wh, bh)

    res, off = [], 0
    for z in hz:
        res.append(outs[:, off:off + z].reshape(B, NP, z))
        off += z
    return (*res, stats)


# speedup vs baseline: 3.4075x; 2.4979x over previous
import jax
import jax.numpy as jnp
from jax import lax
from jax.experimental import pallas as pl
from jax.experimental.pallas import tpu as pltpu

B, T, CIN = 8, 2048, 1
H, L, E, K, NH, F, PL_ = 1024, 2, 8, 2, 16, 1024, 32
NP = T // PL_
N = B * NP
CAP = int(1.25 * N * K / E)
DH = H // NH
HZ = [1, 8, 32, 64]


def _rms(x, w):
    return x * w * lax.rsqrt(jnp.mean(x * x, axis=-1, keepdims=True) + 1e-6)


def _silu(x):
    return x * (1.0 / (1.0 + jnp.exp(-x)))


def _mega_body(p_ref, embW_ref, gateW_ref, ln1_ref, ln2_ref, wr_ref,
               wh_ref, bh_ref,
               qkvW_hbm, oW_hbm, w1_hbm, w2_hbm,
               outs_ref, stats_ref,
               qkvbuf, obuf, w1buf, w2buf,
               sem_qkv, sem_o, sem_w1, sem_w2):

    def qkv_copy(l):
        return pltpu.make_async_copy(qkvW_hbm.at[l], qkvbuf, sem_qkv)

    def o_copy(l):
        return pltpu.make_async_copy(oW_hbm.at[l], obuf, sem_o)

    def w1_copy(l, e):
        return pltpu.make_async_copy(w1_hbm.at[l, e], w1buf.at[e % 3],
                                     sem_w1.at[e % 3])

    def w2_copy(l, e):
        return pltpu.make_async_copy(w2_hbm.at[l, e], w2buf.at[e % 3],
                                     sem_w2.at[e % 3])

    qkv_copy(0).start()
    o_copy(0).start()
    w1_copy(0, 0).start()
    w2_copy(0, 0).start()
    w1_copy(0, 1).start()
    w2_copy(0, 1).start()

    p = p_ref[...]
    h = _silu(p @ gateW_ref[...]) * (p @ embW_ref[...])

    r_iota = lax.broadcasted_iota(jnp.int32, (N, N), 0)
    c_iota = lax.broadcasted_iota(jnp.int32, (N, N), 1)
    mask = ((r_iota // NP) == (c_iota // NP)) & (r_iota >= c_iota)
    lt = (r_iota > c_iota).astype(jnp.float32)
    e_iota = lax.broadcasted_iota(jnp.int32, (N, E), 1)
    c_iota2 = lax.broadcasted_iota(jnp.int32, (N, CAP), 1)

    stats = jnp.zeros((1, E), jnp.float32)

    for l in range(L):
        qkv_copy(l).wait()
        ni = _rms(h, ln1_ref[l:l + 1])
        qkv = ni @ qkvbuf[...]
        ctxs = []
        for hd in range(NH):
            q = qkv[:, hd * DH:(hd + 1) * DH]
            k = qkv[:, H + hd * DH:H + (hd + 1) * DH]
            v = qkv[:, 2 * H + hd * DH:2 * H + (hd + 1) * DH]
            s = lax.dot_general(q, k, (((1,), (1,)), ((), ())))
            s = jnp.where(mask, s * (1.0 / (DH ** 0.5)), -1e9)
            s = s - jnp.max(s, axis=-1, keepdims=True)
            pe = jnp.exp(s)
            a = pe / jnp.sum(pe, axis=-1, keepdims=True)
            ctxs.append(jnp.dot(a, v))
        ctx = jnp.concatenate(ctxs, axis=-1)
        o_copy(l).wait()
        attn = ctx @ obuf[...]
        hs = _rms(attn + ni, ln2_ref[l:l + 1])

        logits = hs @ wr_ref[l]
        m = jnp.max(logits, axis=-1, keepdims=True)
        ex = jnp.exp(logits - m)
        probs = ex / jnp.sum(ex, axis=-1, keepdims=True)
        m0 = jnp.max(probs, axis=-1, keepdims=True)
        i0 = jnp.min(jnp.where(probs == m0, e_iota, E), axis=-1, keepdims=True)
        oh0 = (e_iota == i0).astype(jnp.float32)
        probs1 = jnp.where(e_iota == i0, -1.0, probs)
        m1 = jnp.max(probs1, axis=-1, keepdims=True)
        i1 = jnp.min(jnp.where(probs1 == m1, e_iota, E), axis=-1, keepdims=True)
        oh1 = (e_iota == i1).astype(jnp.float32)
        denom = m0 + m1 + 1e-9
        g0 = m0 / denom
        g1 = m1 / denom
        cum0 = jnp.dot(lt, oh0)
        pos0 = jnp.sum(cum0 * oh0, axis=-1, keepdims=True)
        total0 = jnp.sum(oh0, axis=0, keepdims=True)
        cum1 = jnp.dot(lt, oh1) + total0
        pos1 = jnp.sum(cum1 * oh1, axis=-1, keepdims=True)
        keep0 = (pos0 < CAP).astype(jnp.float32)
        keep1 = (pos1 < CAP).astype(jnp.float32)
        stats = stats + jnp.sum(keep0 * oh0 + keep1 * oh1, axis=0,
                                keepdims=True)
        kg0 = keep0 * g0
        kg1 = keep1 * g1
        pos0i = pos0.astype(jnp.int32)
        pos1i = pos1.astype(jnp.int32)

        y = None
        for e in range(E):
            if e + 2 < E:
                w1_copy(l, e + 2).start()
                w2_copy(l, e + 2).start()
            elif e + 2 == E and l + 1 < L:
                qkv_copy(l + 1).start()
                o_copy(l + 1).start()
            w1_copy(l, e).wait()
            w2_copy(l, e).wait()
            d0 = ((i0 == e) & (pos0i == c_iota2)).astype(jnp.float32)
            d1 = ((i1 == e) & (pos1i == c_iota2)).astype(jnp.float32)
            disp = d0 + d1
            x_e = lax.dot_general(disp, hs, (((0,), (0,)), ((), ())))
            hmid = _silu(x_e @ w1buf[e % 3])
            eout = hmid @ w2buf[e % 3]
            comb = kg0 * d0 + kg1 * d1
            ye = jnp.dot(comb, eout)
            y = ye if y is None else y + ye
        if l + 1 < L:
            w1_copy(l + 1, 0).start()
            w2_copy(l + 1, 0).start()
            w1_copy(l + 1, 1).start()
            w2_copy(l + 1, 1).start()
        h = 2.0 * hs + y

    outs_ref[...] = h @ wh_ref[...] + bh_ref[...]
    stats_ref[...] = stats


def kernel(x, emb_W, gate_W, ln1_w, qkv_W, o_W, ln2_w, router_W,
           exp_W1, exp_W2, hW1, hb1, hW8, hb8, hW32, hb32, hW64, hb64):
    f32 = jnp.float32
    p = x.reshape(B, NP, PL_, CIN).transpose(0, 1, 3, 2).reshape(N, CIN * PL_)
    Wh = jnp.concatenate([hW1, hW8, hW32, hW64], axis=1)
    bh = jnp.concatenate([hb1, hb8, hb32, hb64]).reshape(1, -1)

    vmem = pl.BlockSpec(memory_space=pltpu.MemorySpace.HBM)
    outs, stats = pl.pallas_call(
        _mega_body,
        in_specs=[pl.BlockSpec((N, CIN * PL_), lambda: (0, 0)),
                  pl.BlockSpec((CIN * PL_, H), lambda: (0, 0)),
                  pl.BlockSpec((CIN * PL_, H), lambda: (0, 0)),
                  pl.BlockSpec((L, H), lambda: (0, 0)),
                  pl.BlockSpec((L, H), lambda: (0, 0)),
                  pl.BlockSpec((L, H, E), lambda: (0, 0, 0)),
                  pl.BlockSpec((H, sum(HZ)), lambda: (0, 0)),
                  pl.BlockSpec((1, sum(HZ)), lambda: (0, 0)),
                  vmem, vmem, vmem, vmem],
        out_shape=(jax.ShapeDtypeStruct((N, sum(HZ)), f32),
                   jax.ShapeDtypeStruct((1, E), f32)),
        scratch_shapes=[
            pltpu.VMEM((H, 3 * H), f32),
            pltpu.VMEM((H, H), f32),
            pltpu.VMEM((3, H, F), f32),
            pltpu.VMEM((3, F, H), f32),
            pltpu.SemaphoreType.DMA,
            pltpu.SemaphoreType.DMA,
            pltpu.SemaphoreType.DMA((3,)),
            pltpu.SemaphoreType.DMA((3,)),
        ],
        compiler_params=pltpu.CompilerParams(
            vmem_limit_bytes=100 * 1024 * 1024),
    )(p, emb_W, gate_W, ln1_w, ln2_w, router_W, Wh, bh,
      qkv_W, o_W, exp_W1, exp_W2)

    res, off = [], 0
    for z in HZ:
        res.append(outs[:, off:off + z].reshape(B, NP, z))
        off += z
    return (*res, stats[0])
